# baseline (speedup 1.0000x reference)
import functools

import jax
import jax.numpy as jnp
from jax import lax
from jax.experimental import pallas as pl
from jax.experimental.pallas import tpu as pltpu
from jax.experimental.pallas import tpu_sc as plsc

NO_CAT = 1000000
EMBED_DIM = 32
BATCH = 16384

_L = 16
_TAIL_COL0 = (NO_CAT // 128 - 2) * 128
_TAIL_W = NO_CAT - _TAIL_COL0

_info = plsc.get_sparse_core_info()
_NC = _info.num_cores
_NS = _info.num_subcores
_NW = _NC * _NS

_N_SC = 12288
_N_TC = BATCH - _N_SC
_B_W = _N_SC // _NW
_NCHUNK = _B_W // _L
_TCG = 8

_mesh = plsc.VectorSubcoreMesh(core_axis_name="c", subcore_axis_name="s")


@functools.partial(
    pl.kernel,
    mesh=_mesh,
    out_type=jax.ShapeDtypeStruct((EMBED_DIM, _N_SC), jnp.float32),
    compiler_params=pltpu.CompilerParams(needs_layout_passes=False),
    scratch_types=[
        pltpu.VMEM((_B_W,), jnp.int32),
        pltpu.VMEM((_L, EMBED_DIM, 128), jnp.float32),
        pltpu.VMEM((EMBED_DIM, _B_W), jnp.float32),
        pltpu.VMEM((EMBED_DIM, _TAIL_W), jnp.float32),
        pltpu.SemaphoreType.DMA,
        pltpu.SemaphoreType.DMA,
    ],
)
def _embed_gather_sc(x_hbm, tabt_hbm, outt_hbm, idx_v, tiles_v, out_v, tail_v,
                     sem, tail_sem):
    wid = lax.axis_index("s") * _NC + lax.axis_index("c")
    b0 = wid * _B_W

    tail_cp = pltpu.async_copy(
        tabt_hbm.at[:, pl.ds(_TAIL_COL0, _TAIL_W)], tail_v, tail_sem)
    pltpu.sync_copy(x_hbm.at[pl.ds(b0, _B_W)], idx_v)
    tail_cp.wait()

    iota = lax.iota(jnp.int32, _L)

    def chunk_body(k, _):
        ivec = idx_v[pl.ds(k * _L, _L)]
        tmask = ivec >= _TAIL_COL0
        isafe = jnp.where(tmask, 0, ivec)
        copies = []
        for l in range(_L):
            g = lax.reduce_max(jnp.where(iota == l, isafe, 0), (0,))
            col = pl.multiple_of(lax.shift_left(
                lax.shift_right_logical(g, 7), 7), 128)
            copies.append(pltpu.async_copy(
                tabt_hbm.at[:, pl.ds(col, 128)], tiles_v.at[l], sem))
        for cp in copies:
            cp.wait()
        lane = isafe & 127
        tcol = jnp.maximum(ivec - _TAIL_COL0, 0)
        for d in range(EMBED_DIM):
            drow = jnp.full((_L,), d, jnp.int32)
            vals = plsc.load_gather(tiles_v, [iota, drow, lane])
            tvals = plsc.load_gather(tail_v, [drow, tcol], mask=tmask)
            out_v[d, pl.ds(k * _L, _L)] = jnp.where(tmask, tvals, vals)
        return 0

    lax.fori_loop(0, _NCHUNK, chunk_body, 0)

    pltpu.sync_copy(out_v, outt_hbm.at[:, pl.ds(b0, _B_W)])


def _tc_body(xpref, *refs):
    blocks = refs[:_TCG]
    out_ref = refs[_TCG]
    step = pl.program_id(0)
    lane_iota = lax.broadcasted_iota(jnp.int32, (EMBED_DIM, 128), 1)
    grp = lax.rem(step, 128 // _TCG)
    for j in range(_TCG):
        idx = xpref[step * _TCG + j]
        sel = lane_iota == (idx & 127)
        colvec = jnp.sum(jnp.where(sel, blocks[j][...], 0.0), axis=1,
                         keepdims=True)
        dstmask = lane_iota == grp * _TCG + j
        out_ref[...] = jnp.where(dstmask, colvec, out_ref[...])


_tc_grid_spec = pltpu.PrefetchScalarGridSpec(
    num_scalar_prefetch=1,
    grid=(_N_TC // _TCG,),
    in_specs=[
        pl.BlockSpec(
            (EMBED_DIM, 128),
            functools.partial(
                (lambda j, i, xref: (0, lax.shift_right_logical(
                    xref[i * _TCG + j], 7))), j))
        for j in range(_TCG)
    ],
    out_specs=pl.BlockSpec(
        (EMBED_DIM, 128), lambda i, xref: (0, i // (128 // _TCG))),
)

_embed_gather_tc = pl.pallas_call(
    _tc_body,
    grid_spec=_tc_grid_spec,
    out_shape=jax.ShapeDtypeStruct((EMBED_DIM, _N_TC), jnp.float32),
)


def kernel(x, table):
    xi = x.astype(jnp.int32)
    tabt = table.T
    outt_sc = _embed_gather_sc(xi[:_N_SC], tabt)
    outt_tc = _embed_gather_tc(xi[_N_SC:], *([tabt] * _TCG))
    return jnp.concatenate([outt_sc, outt_tc], axis=1).T


# speedup vs baseline: 2.3537x; 2.3537x over previous
import functools

import jax
import jax.numpy as jnp
from jax import lax
from jax.experimental import pallas as pl
from jax.experimental.pallas import tpu as pltpu
from jax.experimental.pallas import tpu_sc as plsc

NO_CAT = 1000000
EMBED_DIM = 32
BATCH = 16384

_L = 16
_TAIL_COL0 = (NO_CAT // 128 - 2) * 128
_TAIL_W = NO_CAT - _TAIL_COL0

_info = plsc.get_sparse_core_info()
_NC = _info.num_cores
_NS = _info.num_subcores
_NW = _NC * _NS

_N_SC = BATCH
_B_W = _N_SC // _NW
_NCHUNK = _B_W // _L

_mesh = plsc.VectorSubcoreMesh(core_axis_name="c", subcore_axis_name="s")


@functools.partial(
    pl.kernel,
    mesh=_mesh,
    out_type=jax.ShapeDtypeStruct((EMBED_DIM, _N_SC), jnp.float32),
    compiler_params=pltpu.CompilerParams(needs_layout_passes=False),
    scratch_types=[
        pltpu.VMEM((_B_W,), jnp.int32),
        pltpu.VMEM((_L, EMBED_DIM, 128), jnp.float32),
        pltpu.VMEM((EMBED_DIM, _B_W), jnp.float32),
        pltpu.VMEM((EMBED_DIM, _TAIL_W), jnp.float32),
        pltpu.SemaphoreType.DMA,
        pltpu.SemaphoreType.DMA,
    ],
)
def _embed_gather_sc(x_hbm, tabt_hbm, outt_hbm, idx_v, tiles_v, out_v, tail_v,
                     sem, tail_sem):
    wid = lax.axis_index("s") * _NC + lax.axis_index("c")
    b0 = wid * _B_W

    tail_cp = pltpu.async_copy(
        tabt_hbm.at[:, pl.ds(_TAIL_COL0, _TAIL_W)], tail_v, tail_sem)
    pltpu.sync_copy(x_hbm.at[pl.ds(b0, _B_W)], idx_v)
    tail_cp.wait()

    iota = lax.iota(jnp.int32, _L)

    def chunk_body(k, _):
        ivec = idx_v[pl.ds(k * _L, _L)]
        tmask = ivec >= _TAIL_COL0
        isafe = jnp.where(tmask, 0, ivec)
        copies = []
        for l in range(_L):
            g = lax.reduce_max(jnp.where(iota == l, isafe, 0), (0,))
            col = pl.multiple_of(lax.shift_left(
                lax.shift_right_logical(g, 7), 7), 128)
            copies.append(pltpu.async_copy(
                tabt_hbm.at[:, pl.ds(col, 128)], tiles_v.at[l], sem))
        for cp in copies:
            cp.wait()
        lane = isafe & 127
        tcol = jnp.maximum(ivec - _TAIL_COL0, 0)
        for d in range(EMBED_DIM):
            drow = jnp.full((_L,), d, jnp.int32)
            vals = plsc.load_gather(tiles_v, [iota, drow, lane])
            tvals = plsc.load_gather(tail_v, [drow, tcol], mask=tmask)
            out_v[d, pl.ds(k * _L, _L)] = jnp.where(tmask, tvals, vals)
        return 0

    lax.fori_loop(0, _NCHUNK, chunk_body, 0)

    pltpu.sync_copy(out_v, outt_hbm.at[:, pl.ds(b0, _B_W)])


def kernel(x, table):
    outt = _embed_gather_sc(x.astype(jnp.int32), table.T)
    return outt.T
